# Initial kernel scaffold; baseline (speedup 1.0000x reference)
import jax
import jax.numpy as jnp
from jax.experimental import pallas as pl


def kernel(points, queries, prev_features, W1, b1, W2, b2, Wc, bc):
    raise NotImplementedError("write your pallas kernel here")


# speedup vs baseline: 2.9622x; 2.9622x over previous
import functools

import jax
import jax.numpy as jnp
from jax import lax
from jax.experimental import pallas as pl
from jax.experimental.pallas import tpu as pltpu
from jax.experimental.pallas import tpu_sc as plsc

K = 64
DIV = 8
S = K // DIV

NC = 2
NS = 16
NW = NC * NS

MB = 256


def _knn_mlp_body(q_ref, p_ref, w1_ref, b1_ref, w2_ref, b2_ref,
                  idx_ref, hp_ref):
    b = pl.program_id(0)
    q = q_ref[0]
    p = p_ref[0]
    n = p.shape[0]
    q2 = jnp.sum(q * q, axis=1, keepdims=True)
    p2 = jnp.sum(p * p, axis=1)[None, :]
    inner = lax.dot_general(q, p, (((1,), (1,)), ((), ())),
                            preferred_element_type=jnp.float32)
    dist = q2 - 2.0 * inner + p2

    iota_n = lax.broadcasted_iota(jnp.int32, (MB, n), 1)
    iota_k = lax.broadcasted_iota(jnp.int32, (MB, K), 1)
    iota_k3 = lax.broadcasted_iota(jnp.int32, (MB, K, 3), 1)

    def body(k, carry):
        d, idxacc, locacc = carry
        m = jnp.min(d, axis=1, keepdims=True)
        eq = d == m
        am = jnp.min(jnp.where(eq, iota_n, n), axis=1)
        onehot = iota_n == am[:, None]
        sel = lax.dot_general(onehot.astype(jnp.float32), p,
                              (((1,), (0,)), ((), ())),
                              preferred_element_type=jnp.float32)
        loc = q - sel
        sel_k = iota_k == k
        idxacc = jnp.where(sel_k, am[:, None] + b * n, idxacc)
        locacc = jnp.where(iota_k3 == k, loc[:, None, :], locacc)
        d = jnp.where(onehot, jnp.inf, d)
        return d, idxacc, locacc

    idx0 = jnp.zeros((MB, K), jnp.int32)
    loc0 = jnp.zeros((MB, K, 3), jnp.float32)
    _, idxacc, locacc = lax.fori_loop(0, K, body, (dist, idx0, loc0))

    idx_ref[0] = idxacc
    loc2 = locacc.reshape(MB * K, 3)
    h = jnp.maximum(lax.dot_general(loc2, w1_ref[...],
                                    (((1,), (0,)), ((), ())),
                                    preferred_element_type=jnp.float32)
                    + b1_ref[...], 0.0)
    h = jnp.maximum(lax.dot_general(h, w2_ref[...],
                                    (((1,), (0,)), ((), ())),
                                    preferred_element_type=jnp.float32)
                    + b2_ref[...], 0.0)
    hp_ref[0] = jnp.max(h.reshape(MB, DIV, S, 64), axis=2)


def _tc_knn_mlp(queries, points, W1, b1, W2, b2):
    B, M, _ = queries.shape
    N = points.shape[1]
    grid = (B, M // MB)
    return pl.pallas_call(
        _knn_mlp_body,
        grid=grid,
        in_specs=[
            pl.BlockSpec((1, MB, 3), lambda b, i: (b, i, 0)),
            pl.BlockSpec((1, N, 3), lambda b, i: (b, 0, 0)),
            pl.BlockSpec((3, 32), lambda b, i: (0, 0)),
            pl.BlockSpec((1, 32), lambda b, i: (0, 0)),
            pl.BlockSpec((32, 64), lambda b, i: (0, 0)),
            pl.BlockSpec((1, 64), lambda b, i: (0, 0)),
        ],
        out_specs=[
            pl.BlockSpec((1, MB, K), lambda b, i: (b, i, 0)),
            pl.BlockSpec((1, MB, DIV, 64), lambda b, i: (b, i, 0, 0)),
        ],
        out_shape=[
            jax.ShapeDtypeStruct((B, M, K), jnp.int32),
            jax.ShapeDtypeStruct((B, M, DIV, 64), jnp.float32),
        ],
    )(queries, points, W1, b1.reshape(1, 32), W2, b2.reshape(1, 64))


CH = 512
CW = CH // S


def _sc_gather_pool(table, idx):
    R = idx.shape[0]
    F = table.shape[1]
    rows_per_w = R // NW
    n_chunks = rows_per_w // CH
    mesh = plsc.VectorSubcoreMesh(core_axis_name="c", subcore_axis_name="s")

    @functools.partial(
        pl.kernel,
        mesh=mesh,
        out_type=jax.ShapeDtypeStruct((R // S, F), jnp.float32),
        scratch_types=[
            pltpu.VMEM((CH,), jnp.int32),
            pltpu.VMEM((CW, S, F), jnp.float32),
            pltpu.VMEM((CW, F), jnp.float32),
            pltpu.SemaphoreType.DMA,
        ],
    )
    def k(table_hbm, idx_hbm, out_hbm, idx_v, rows_v, pool_v, sem):
        wid = lax.axis_index("s") * NC + lax.axis_index("c")
        base = wid * rows_per_w

        def chunk_body(i, _):
            row0 = pl.multiple_of(base + i * CH, CH)
            pltpu.sync_copy(idx_hbm.at[pl.ds(row0, CH)], idx_v)
            rv = rows_v.reshape(CH, F)
            pltpu.async_copy(table_hbm.at[idx_v], rv, sem).wait()

            def win_body(w, _):
                for c in range(F // 16):
                    acc = rows_v[w, 0, pl.ds(c * 16, 16)]
                    for r in range(1, S):
                        acc = jnp.maximum(acc, rows_v[w, r, pl.ds(c * 16, 16)])
                    pool_v[w, pl.ds(c * 16, 16)] = acc
                return 0

            lax.fori_loop(0, CW, win_body, 0)
            pout = pl.multiple_of((base + i * CH) // S, CW)
            pltpu.sync_copy(pool_v, out_hbm.at[pl.ds(pout, CW)])
            return 0

        lax.fori_loop(0, n_chunks, chunk_body, 0)

    return k(table, idx)


def _conv_body(hp_ref, fp_ref, wh_ref, wf_ref, bc_ref, out_ref):
    mb = hp_ref.shape[0]
    acc = jnp.broadcast_to(bc_ref[...], (mb, bc_ref.shape[1]))
    hp = hp_ref[...]
    fp = fp_ref[...].reshape(mb, DIV, fp_ref.shape[1])
    for d in range(DIV):
        acc = acc + lax.dot_general(hp[:, d, :], wh_ref[d],
                                    (((1,), (0,)), ((), ())),
                                    preferred_element_type=jnp.float32)
        acc = acc + lax.dot_general(fp[:, d, :], wf_ref[d],
                                    (((1,), (0,)), ((), ())),
                                    preferred_element_type=jnp.float32)
    out_ref[...] = jnp.maximum(acc, 0.0)


def _tc_conv(hp, fp, Wh, Wf, bc):
    BM = hp.shape[0]
    F = fp.shape[1]
    O = bc.shape[0]
    QB = 512
    grid = (BM // QB,)
    return pl.pallas_call(
        _conv_body,
        grid=grid,
        in_specs=[
            pl.BlockSpec((QB, DIV, 64), lambda i: (i, 0, 0)),
            pl.BlockSpec((QB * DIV, F), lambda i: (i, 0)),
            pl.BlockSpec((DIV, 64, O), lambda i: (0, 0, 0)),
            pl.BlockSpec((DIV, F, O), lambda i: (0, 0, 0)),
            pl.BlockSpec((1, O), lambda i: (0, 0)),
        ],
        out_specs=pl.BlockSpec((QB, O), lambda i: (i, 0)),
        out_shape=jax.ShapeDtypeStruct((BM, O), jnp.float32),
    )(hp, fp, Wh, Wf, bc.reshape(1, O))


def kernel(points, queries, prev_features, W1, b1, W2, b2, Wc, bc):
    B, N, _ = points.shape
    M = queries.shape[1]
    F1 = prev_features.shape[2]
    O = bc.shape[0]

    idx, hp = _tc_knn_mlp(queries, points, W1, b1, W2, b2)

    table = prev_features.reshape(B * N, F1)
    fp = _sc_gather_pool(table, idx.reshape(B * M * K))

    Wh = jnp.transpose(Wc[:, :64, 0, :], (2, 1, 0))
    Wf = jnp.transpose(Wc[:, 64:, 0, :], (2, 1, 0))

    out = _tc_conv(hp.reshape(B * M, DIV, 64), fp, Wh, Wf, bc)
    return out.reshape(B, M, O)


# speedup vs baseline: 5.2208x; 1.7624x over previous
import functools

import jax
import jax.numpy as jnp
from jax import lax
from jax.experimental import pallas as pl
from jax.experimental.pallas import tpu as pltpu
from jax.experimental.pallas import tpu_sc as plsc

K = 64
DIV = 8
S = K // DIV
P4 = 4

NC = 2
NS = 16
NW = NC * NS

MB = 256


def _knn_body(q_ref, p_ref, idx_ref):
    b = pl.program_id(0)
    q = q_ref[0]
    p = p_ref[0]
    n = p.shape[0]
    q2 = jnp.sum(q * q, axis=1, keepdims=True)
    p2 = jnp.sum(p * p, axis=1)[None, :]
    inner = lax.dot_general(q, p, (((1,), (1,)), ((), ())),
                            preferred_element_type=jnp.float32)
    dist = q2 - 2.0 * inner + p2

    iota_n = lax.broadcasted_iota(jnp.int32, (MB, n), 1)
    iota_k = lax.broadcasted_iota(jnp.int32, (MB, K), 1)

    def body(k, carry):
        d, idxacc = carry
        m = jnp.min(d, axis=1, keepdims=True)
        am = jnp.min(jnp.where(d == m, iota_n, n), axis=1)
        idxacc = jnp.where(iota_k == k, am[:, None] + b * n, idxacc)
        d = jnp.where(iota_n == am[:, None], jnp.inf, d)
        return d, idxacc

    idx0 = jnp.zeros((MB, K), jnp.int32)
    _, idxacc = lax.fori_loop(0, K, body, (dist, idx0))
    idx_ref[0] = idxacc


def _tc_knn(queries, points):
    B, M, _ = queries.shape
    N = points.shape[1]
    grid = (B, M // MB)
    return pl.pallas_call(
        _knn_body,
        grid=grid,
        in_specs=[
            pl.BlockSpec((1, MB, 3), lambda b, i: (b, i, 0)),
            pl.BlockSpec((1, N, 3), lambda b, i: (b, 0, 0)),
        ],
        out_specs=pl.BlockSpec((1, MB, K), lambda b, i: (b, i, 0)),
        out_shape=jax.ShapeDtypeStruct((B, M, K), jnp.int32),
    )(queries, points)


CH = 512
CW = CH // S


def _sc_gather(ptab1, ftab, idx):
    R = idx.shape[0]
    F = ftab.shape[1]
    rows_per_w = R // NW
    n_chunks = rows_per_w // CH
    mesh = plsc.VectorSubcoreMesh(core_axis_name="c", subcore_axis_name="s")

    @functools.partial(
        pl.kernel,
        mesh=mesh,
        out_type=[
            jax.ShapeDtypeStruct((R * P4,), jnp.float32),
            jax.ShapeDtypeStruct((R // S, F), jnp.float32),
        ],
        scratch_types=[
            pltpu.VMEM((CH,), jnp.int32),
            pltpu.VMEM((CH * P4,), jnp.int32),
            pltpu.VMEM((CH * P4,), jnp.float32),
            pltpu.VMEM((CW, S, F), jnp.float32),
            pltpu.VMEM((CW, F), jnp.float32),
            pltpu.SemaphoreType.DMA,
            pltpu.SemaphoreType.DMA,
        ],
    )
    def k(ptab_hbm, ftab_hbm, idx_hbm, pout_hbm, fout_hbm,
          idx_v, eidx_v, prow_v, frow_v, pool_v, psem, fsem):
        wid = lax.axis_index("s") * NC + lax.axis_index("c")
        base = wid * rows_per_w
        iota16 = lax.broadcasted_iota(jnp.int32, (16,), 0)

        def chunk_body(i, _):
            row0 = pl.multiple_of(base + i * CH, CH)
            pltpu.sync_copy(idx_hbm.at[pl.ds(row0, CH)], idx_v)

            def eidx_body(g, _):
                g0 = pl.multiple_of(g * 16, 16)
                v4 = idx_v[pl.ds(g0, 16)] * P4
                for c in range(P4):
                    eidx_v[pl.ds(c * CH + g0, 16)] = v4 + c
                return 0

            lax.fori_loop(0, CH // 16, eidx_body, 0)
            pcopy = pltpu.async_copy(ptab_hbm.at[eidx_v], prow_v, psem)
            fv = frow_v.reshape(CH, F)
            pltpu.async_copy(ftab_hbm.at[idx_v], fv, fsem).wait()

            def win_body(w, _):
                for c in range(F // 16):
                    acc = frow_v[w, 0, pl.ds(c * 16, 16)]
                    for r in range(1, S):
                        acc = jnp.maximum(acc, frow_v[w, r, pl.ds(c * 16, 16)])
                    pool_v[w, pl.ds(c * 16, 16)] = acc
                return 0

            lax.fori_loop(0, CW, win_body, 0)
            fo = pl.multiple_of((base + i * CH) // S, CW)
            pltpu.sync_copy(pool_v, fout_hbm.at[pl.ds(fo, CW)])

            pcopy.wait()
            po = pl.multiple_of((base + i * CH) * P4, CH * P4)
            pltpu.sync_copy(prow_v, pout_hbm.at[pl.ds(po, CH * P4)])
            return 0

        lax.fori_loop(0, n_chunks, chunk_body, 0)

    return k(ptab1, ftab, idx)


def _mlp_conv_body(q_ref, p_ref, fp_ref, w1_ref, b1_ref, w2_ref, b2_ref,
                   wh_ref, wf_ref, bc_ref, out_ref):
    mb = q_ref.shape[0]
    prow = p_ref[...].reshape(mb, K, P4)
    loc = q_ref[...][:, None, :] - prow
    h = jnp.maximum(lax.dot_general(loc.reshape(mb * K, P4), w1_ref[...],
                                    (((1,), (0,)), ((), ())),
                                    preferred_element_type=jnp.float32)
                    + b1_ref[...], 0.0)
    h = jnp.maximum(lax.dot_general(h, w2_ref[...],
                                    (((1,), (0,)), ((), ())),
                                    preferred_element_type=jnp.float32)
                    + b2_ref[...], 0.0)
    hp = jnp.max(h.reshape(mb, DIV, S, 64), axis=2)

    acc = jnp.broadcast_to(bc_ref[...], (mb, bc_ref.shape[1]))
    fp = fp_ref[...].reshape(mb, DIV, fp_ref.shape[1])
    for d in range(DIV):
        acc = acc + lax.dot_general(hp[:, d, :], wh_ref[d],
                                    (((1,), (0,)), ((), ())),
                                    preferred_element_type=jnp.float32)
        acc = acc + lax.dot_general(fp[:, d, :], wf_ref[d],
                                    (((1,), (0,)), ((), ())),
                                    preferred_element_type=jnp.float32)
    out_ref[...] = jnp.maximum(acc, 0.0)


def _tc_mlp_conv(qpad, prows, fp, W1p, b1, W2, b2, Wh, Wf, bc):
    BM = qpad.shape[0]
    F = fp.shape[1]
    O = bc.shape[0]
    QB = 512
    grid = (BM // QB,)
    return pl.pallas_call(
        _mlp_conv_body,
        grid=grid,
        in_specs=[
            pl.BlockSpec((QB, P4), lambda i: (i, 0)),
            pl.BlockSpec((QB * K, P4), lambda i: (i, 0)),
            pl.BlockSpec((QB * DIV, F), lambda i: (i, 0)),
            pl.BlockSpec((P4, 32), lambda i: (0, 0)),
            pl.BlockSpec((1, 32), lambda i: (0, 0)),
            pl.BlockSpec((32, 64), lambda i: (0, 0)),
            pl.BlockSpec((1, 64), lambda i: (0, 0)),
            pl.BlockSpec((DIV, 64, O), lambda i: (0, 0, 0)),
            pl.BlockSpec((DIV, F, O), lambda i: (0, 0, 0)),
            pl.BlockSpec((1, O), lambda i: (0, 0)),
        ],
        out_specs=pl.BlockSpec((QB, O), lambda i: (i, 0)),
        out_shape=jax.ShapeDtypeStruct((BM, O), jnp.float32),
    )(qpad, prows, fp, W1p, b1.reshape(1, 32), W2, b2.reshape(1, 64),
      Wh, Wf, bc.reshape(1, O))


def kernel(points, queries, prev_features, W1, b1, W2, b2, Wc, bc):
    B, N, _ = points.shape
    M = queries.shape[1]
    F1 = prev_features.shape[2]
    O = bc.shape[0]

    idx = _tc_knn(queries, points)

    ptab1 = jnp.pad(points.reshape(B * N, 3),
                    ((0, 0), (0, P4 - 3))).reshape(B * N * P4)
    ftab = prev_features.reshape(B * N, F1)
    pflat, fp = _sc_gather(ptab1, ftab, idx.reshape(B * M * K))
    R = B * M * K
    prows = jnp.transpose(pflat.reshape(R // CH, P4, CH), (0, 2, 1))
    prows = prows.reshape(R, P4)

    qpad = jnp.pad(queries.reshape(B * M, 3), ((0, 0), (0, P4 - 3)))
    W1p = jnp.pad(W1, ((0, P4 - 3), (0, 0)))

    Wh = jnp.transpose(Wc[:, :64, 0, :], (2, 1, 0))
    Wf = jnp.transpose(Wc[:, 64:, 0, :], (2, 1, 0))

    out = _tc_mlp_conv(qpad, prows, fp, W1p, b1, W2, b2, Wh, Wf, bc)
    return out.reshape(B, M, O)


# speedup vs baseline: 5.3530x; 1.0253x over previous
import functools

import jax
import jax.numpy as jnp
from jax import lax
from jax.experimental import pallas as pl
from jax.experimental.pallas import tpu as pltpu
from jax.experimental.pallas import tpu_sc as plsc

K = 64
DIV = 8
S = K // DIV
P4 = 4

NC = 2
NS = 16
NW = NC * NS

MB = 512


def _knn_body(q_ref, p_ref, idx_ref):
    b = pl.program_id(0)
    q = q_ref[0]
    p = p_ref[0]
    n = p.shape[0]
    q2 = jnp.sum(q * q, axis=1, keepdims=True)
    p2 = jnp.sum(p * p, axis=1)[None, :]
    inner = lax.dot_general(q, p, (((1,), (1,)), ((), ())),
                            preferred_element_type=jnp.float32)
    dist = q2 - 2.0 * inner + p2

    iota_n = lax.broadcasted_iota(jnp.int32, (MB, n), 1)
    iota_k = lax.broadcasted_iota(jnp.int32, (MB, K), 1)

    def body(k, carry):
        d, idxacc = carry
        m = jnp.min(d, axis=1, keepdims=True)
        am = jnp.min(jnp.where(d == m, iota_n, n), axis=1)
        idxacc = jnp.where(iota_k == k, am[:, None] + b * n, idxacc)
        d = jnp.where(iota_n == am[:, None], jnp.inf, d)
        return d, idxacc

    idx0 = jnp.zeros((MB, K), jnp.int32)
    _, idxacc = lax.fori_loop(0, K, body, (dist, idx0))
    idx_ref[0] = idxacc


def _tc_knn(queries, points):
    B, M, _ = queries.shape
    N = points.shape[1]
    grid = (B, M // MB)
    return pl.pallas_call(
        _knn_body,
        grid=grid,
        in_specs=[
            pl.BlockSpec((1, MB, 3), lambda b, i: (b, i, 0)),
            pl.BlockSpec((1, N, 3), lambda b, i: (b, 0, 0)),
        ],
        out_specs=pl.BlockSpec((1, MB, K), lambda b, i: (b, i, 0)),
        out_shape=jax.ShapeDtypeStruct((B, M, K), jnp.int32),
    )(queries, points)


CH = 512
CW = CH // S


def _sc_gather(ptab1, ftab, idx):
    R = idx.shape[0]
    F = ftab.shape[1]
    rows_per_w = R // NW
    n_chunks = rows_per_w // CH
    mesh = plsc.VectorSubcoreMesh(core_axis_name="c", subcore_axis_name="s")

    @functools.partial(
        pl.kernel,
        mesh=mesh,
        out_type=[
            jax.ShapeDtypeStruct((R * P4,), jnp.float32),
            jax.ShapeDtypeStruct((R // S, F), jnp.float32),
        ],
        scratch_types=[
            pltpu.VMEM((CH,), jnp.int32),
            pltpu.VMEM((CH * P4,), jnp.int32),
            pltpu.VMEM((CH * P4,), jnp.float32),
            pltpu.VMEM((CW, S, F), jnp.float32),
            pltpu.VMEM((CW, F), jnp.float32),
            pltpu.SemaphoreType.DMA,
            pltpu.SemaphoreType.DMA,
        ],
    )
    def k(ptab_hbm, ftab_hbm, idx_hbm, pout_hbm, fout_hbm,
          idx_v, eidx_v, prow_v, frow_v, pool_v, psem, fsem):
        wid = lax.axis_index("s") * NC + lax.axis_index("c")
        base = wid * rows_per_w
        iota16 = lax.broadcasted_iota(jnp.int32, (16,), 0)

        def chunk_body(i, _):
            row0 = pl.multiple_of(base + i * CH, CH)
            pltpu.sync_copy(idx_hbm.at[pl.ds(row0, CH)], idx_v)

            def eidx_body(g, _):
                g0 = pl.multiple_of(g * 16, 16)
                v4 = idx_v[pl.ds(g0, 16)] * P4
                for c in range(P4):
                    eidx_v[pl.ds(c * CH + g0, 16)] = v4 + c
                return 0

            lax.fori_loop(0, CH // 16, eidx_body, 0)
            pcopy = pltpu.async_copy(ptab_hbm.at[eidx_v], prow_v, psem)
            fv = frow_v.reshape(CH, F)
            pltpu.async_copy(ftab_hbm.at[idx_v], fv, fsem).wait()

            def win_body(w, _):
                for c in range(F // 16):
                    acc = frow_v[w, 0, pl.ds(c * 16, 16)]
                    for r in range(1, S):
                        acc = jnp.maximum(acc, frow_v[w, r, pl.ds(c * 16, 16)])
                    pool_v[w, pl.ds(c * 16, 16)] = acc
                return 0

            lax.fori_loop(0, CW, win_body, 0)
            fo = pl.multiple_of((base + i * CH) // S, CW)
            pltpu.sync_copy(pool_v, fout_hbm.at[pl.ds(fo, CW)])

            pcopy.wait()
            po = pl.multiple_of((base + i * CH) * P4, CH * P4)
            pltpu.sync_copy(prow_v, pout_hbm.at[pl.ds(po, CH * P4)])
            return 0

        lax.fori_loop(0, n_chunks, chunk_body, 0)

    return k(ptab1, ftab, idx)


def _mlp_conv_body(q_ref, p_ref, fp_ref, w1_ref, b1_ref, w2_ref, b2_ref,
                   wh_ref, wf_ref, bc_ref, out_ref):
    mb = q_ref.shape[0]
    prow = p_ref[...].reshape(mb, K, P4)
    loc = q_ref[...][:, None, :] - prow
    h = jnp.maximum(lax.dot_general(loc.reshape(mb * K, P4), w1_ref[...],
                                    (((1,), (0,)), ((), ())),
                                    preferred_element_type=jnp.float32)
                    + b1_ref[...], 0.0)
    h = jnp.maximum(lax.dot_general(h, w2_ref[...],
                                    (((1,), (0,)), ((), ())),
                                    preferred_element_type=jnp.float32)
                    + b2_ref[...], 0.0)
    hp = jnp.max(h.reshape(mb, DIV, S, 64), axis=2)

    acc = jnp.broadcast_to(bc_ref[...], (mb, bc_ref.shape[1]))
    fp = fp_ref[...].reshape(mb, DIV, fp_ref.shape[1])
    for d in range(DIV):
        acc = acc + lax.dot_general(hp[:, d, :], wh_ref[d],
                                    (((1,), (0,)), ((), ())),
                                    preferred_element_type=jnp.float32)
        acc = acc + lax.dot_general(fp[:, d, :], wf_ref[d],
                                    (((1,), (0,)), ((), ())),
                                    preferred_element_type=jnp.float32)
    out_ref[...] = jnp.maximum(acc, 0.0)


def _tc_mlp_conv(qpad, prows, fp, W1p, b1, W2, b2, Wh, Wf, bc):
    BM = qpad.shape[0]
    F = fp.shape[1]
    O = bc.shape[0]
    QB = 512
    grid = (BM // QB,)
    return pl.pallas_call(
        _mlp_conv_body,
        grid=grid,
        in_specs=[
            pl.BlockSpec((QB, P4), lambda i: (i, 0)),
            pl.BlockSpec((QB * K, P4), lambda i: (i, 0)),
            pl.BlockSpec((QB * DIV, F), lambda i: (i, 0)),
            pl.BlockSpec((P4, 32), lambda i: (0, 0)),
            pl.BlockSpec((1, 32), lambda i: (0, 0)),
            pl.BlockSpec((32, 64), lambda i: (0, 0)),
            pl.BlockSpec((1, 64), lambda i: (0, 0)),
            pl.BlockSpec((DIV, 64, O), lambda i: (0, 0, 0)),
            pl.BlockSpec((DIV, F, O), lambda i: (0, 0, 0)),
            pl.BlockSpec((1, O), lambda i: (0, 0)),
        ],
        out_specs=pl.BlockSpec((QB, O), lambda i: (i, 0)),
        out_shape=jax.ShapeDtypeStruct((BM, O), jnp.float32),
    )(qpad, prows, fp, W1p, b1.reshape(1, 32), W2, b2.reshape(1, 64),
      Wh, Wf, bc.reshape(1, O))


def kernel(points, queries, prev_features, W1, b1, W2, b2, Wc, bc):
    B, N, _ = points.shape
    M = queries.shape[1]
    F1 = prev_features.shape[2]
    O = bc.shape[0]

    idx = _tc_knn(queries, points)

    ptab1 = jnp.pad(points.reshape(B * N, 3),
                    ((0, 0), (0, P4 - 3))).reshape(B * N * P4)
    ftab = prev_features.reshape(B * N, F1)
    pflat, fp = _sc_gather(ptab1, ftab, idx.reshape(B * M * K))
    R = B * M * K
    prows = jnp.transpose(pflat.reshape(R // CH, P4, CH), (0, 2, 1))
    prows = prows.reshape(R, P4)

    qpad = jnp.pad(queries.reshape(B * M, 3), ((0, 0), (0, P4 - 3)))
    W1p = jnp.pad(W1, ((0, P4 - 3), (0, 0)))

    Wh = jnp.transpose(Wc[:, :64, 0, :], (2, 1, 0))
    Wf = jnp.transpose(Wc[:, 64:, 0, :], (2, 1, 0))

    out = _tc_mlp_conv(qpad, prows, fp, W1p, b1, W2, b2, Wh, Wf, bc)
    return out.reshape(B, M, O)
